# Initial kernel scaffold; baseline (speedup 1.0000x reference)
import jax
import jax.numpy as jnp
from jax.experimental import pallas as pl


def kernel(tokens, token_embed, pos_embed, attn_W, attn_b, proj_W, proj_b, ln_scale, ln_bias, proj2_W, proj2_b):
    raise NotImplementedError("write your pallas kernel here")


# speedup vs baseline: 10.2194x; 10.2194x over previous
import functools

import jax
import jax.numpy as jnp
from jax import lax
from jax.experimental import pallas as pl
from jax.experimental.pallas import tpu as pltpu
from jax.experimental.pallas import tpu_sc as plsc

B = 16384
SEQ = 50
VOCAB = 100000
EMBED = 64
OUT = 128

NC = 2
NS = 16
NW = NC * NS

_TS_GRID = 25
_TS_ROWS = VOCAB // _TS_GRID


def _tc_scores_body(emb_ref, aw_ref, out_ref):
    i = pl.program_id(0)
    x = emb_ref[...]
    s = jnp.sum(x * aw_ref[...], axis=1)
    out_ref[pl.ds(i, 1), :] = s.reshape(1, _TS_ROWS)


def _tc_scores(token_embed, aw_row):
    out = pl.pallas_call(
        _tc_scores_body,
        grid=(_TS_GRID,),
        in_specs=[
            pl.BlockSpec((_TS_ROWS, EMBED), lambda i: (i, 0)),
            pl.BlockSpec((1, EMBED), lambda i: (0, 0)),
        ],
        out_specs=pl.BlockSpec((_TS_GRID, _TS_ROWS), lambda i: (0, 0)),
        out_shape=jax.ShapeDtypeStruct((_TS_GRID, _TS_ROWS), jnp.float32),
    )(token_embed, aw_row)
    return out.reshape(VOCAB)


_TOK_PER_W = B * SEQ // NW
_A_CHUNK = 6400
_A_NCHUNK = _TOK_PER_W // _A_CHUNK

_sc_mesh = plsc.VectorSubcoreMesh(
    core_axis_name="c", subcore_axis_name="s", num_cores=NC, num_subcores=NS)


@functools.partial(
    pl.kernel,
    out_type=jax.ShapeDtypeStruct((B * SEQ,), jnp.float32),
    mesh=_sc_mesh,
    compiler_params=pltpu.CompilerParams(needs_layout_passes=False, use_tc_tiling_on_sc=False),
    scratch_types=[
        pltpu.VMEM((VOCAB,), jnp.float32),
        pltpu.VMEM((_A_CHUNK,), jnp.int32),
        pltpu.VMEM((_A_CHUNK,), jnp.float32),
    ],
)
def _sc_score_gather(ts_hbm, tok_hbm, out_hbm, table_v, tok_v, out_v):
    wid = lax.axis_index("s") * NC + lax.axis_index("c")
    base = wid * _TOK_PER_W
    pltpu.sync_copy(ts_hbm, table_v)
    for c in range(_A_NCHUNK):
        off = base + c * _A_CHUNK
        pltpu.sync_copy(tok_hbm.at[pl.ds(off, _A_CHUNK)], tok_v)

        def body(i, _):
            idx = tok_v[pl.ds(i * 16, 16)]
            out_v[pl.ds(i * 16, 16)] = plsc.load_gather(table_v, [idx])
            return 0

        lax.fori_loop(0, _A_CHUNK // 16, body, 0)
        pltpu.sync_copy(out_v, out_hbm.at[pl.ds(off, _A_CHUNK)])


_MID_ROWS = 2048
_MID_GRID = B // _MID_ROWS


def _tc_mid_body(rawts_ref, pe_ref, aw_ref, w_ref, ppe_ref):
    pe50 = pe_ref[0:SEQ, :]
    ps = jnp.sum(pe50 * aw_ref[...], axis=1)
    scores = rawts_ref[...] + ps[None, :]
    m = jnp.max(scores, axis=1, keepdims=True)
    e = jnp.exp(scores - m)
    s = jnp.sum(e, axis=1, keepdims=True)
    w = e / s
    w_ref[...] = jnp.concatenate(
        [w, jnp.zeros((w.shape[0], 64 - SEQ), jnp.float32)], axis=1)
    ppe_ref[...] = jnp.dot(w, pe50, preferred_element_type=jnp.float32)


def _tc_mid(rawts, pos_embed, aw_row):
    return pl.pallas_call(
        _tc_mid_body,
        grid=(_MID_GRID,),
        in_specs=[
            pl.BlockSpec((_MID_ROWS, SEQ), lambda i: (i, 0)),
            pl.BlockSpec((64, EMBED), lambda i: (0, 0)),
            pl.BlockSpec((1, EMBED), lambda i: (0, 0)),
        ],
        out_specs=[
            pl.BlockSpec((_MID_ROWS, 64), lambda i: (i, 0)),
            pl.BlockSpec((_MID_ROWS, EMBED), lambda i: (i, 0)),
        ],
        out_shape=[
            jax.ShapeDtypeStruct((B, 64), jnp.float32),
            jax.ShapeDtypeStruct((B, EMBED), jnp.float32),
        ],
    )(rawts, pos_embed, aw_row)


_ROWS_PER_W = B // NW
_G = 16
_NGRP = _ROWS_PER_W // _G


@functools.partial(
    pl.kernel,
    out_type=jax.ShapeDtypeStruct((B, EMBED), jnp.float32),
    mesh=_sc_mesh,
    compiler_params=pltpu.CompilerParams(needs_layout_passes=False, use_tc_tiling_on_sc=False),
    scratch_types=[
        pltpu.VMEM((_G, SEQ), jnp.int32),
        pltpu.VMEM((_G, 64), jnp.float32),
        pltpu.VMEM((_G * SEQ, EMBED), jnp.float32),
        pltpu.VMEM((_G, EMBED), jnp.float32),
        pltpu.SemaphoreType.DMA,
    ],
)
def _sc_pool(table_hbm, tok_hbm, w_hbm, out_hbm, tok_v, w_v, rows_v, out_v,
             gsem):
    wid = lax.axis_index("s") * NC + lax.axis_index("c")
    rbase = wid * _ROWS_PER_W

    def grp(g, _):
        gb = rbase + g * _G
        pltpu.sync_copy(tok_hbm.at[pl.ds(gb, _G)], tok_v)
        pltpu.sync_copy(w_hbm.at[pl.ds(gb, _G)], w_v)
        copies = []
        for r in range(_G):
            copies.append(pltpu.async_copy(
                table_hbm.at[tok_v.at[r]],
                rows_v.at[pl.ds(r * SEQ, SEQ)], gsem))
        for cp in copies:
            cp.wait()

        def row(r, _):
            rb = r * SEQ
            accs = [jnp.zeros((16,), jnp.float32) for _ in range(4)]
            wrows = [w_v[r, pl.ds(k * 16, 16)] for k in range(4)]
            for t in range(SEQ):
                wv = jnp.full((16,), wrows[t // 16][t % 16], jnp.float32)
                for c4 in range(4):
                    accs[c4] = accs[c4] + rows_v[rb + t, pl.ds(c4 * 16, 16)] * wv
            for c4 in range(4):
                out_v[r, pl.ds(c4 * 16, 16)] = accs[c4]
            return 0

        lax.fori_loop(0, _G, row, 0)
        pltpu.sync_copy(out_v, out_hbm.at[pl.ds(gb, _G)])
        return 0

    lax.fori_loop(0, _NGRP, grp, 0)


_FIN_ROWS = 2048
_FIN_GRID = B // _FIN_ROWS


def _tc_final_body(ptok_ref, ppe_ref, pw_ref, pb_ref, lns_ref, lnb_ref,
                   p2w_ref, p2b_ref, out_ref):
    pooled = ptok_ref[...] + ppe_ref[...]
    h = jnp.dot(pooled, pw_ref[...], preferred_element_type=jnp.float32)
    h = h + pb_ref[...]
    mean = jnp.mean(h, axis=-1, keepdims=True)
    d = h - mean
    var = jnp.mean(d * d, axis=-1, keepdims=True)
    h = d / jnp.sqrt(var + 1e-6)
    h = h * lns_ref[...] + lnb_ref[...]
    h = jax.nn.gelu(h)
    out_ref[...] = jnp.dot(h, p2w_ref[...],
                           preferred_element_type=jnp.float32) + p2b_ref[...]


def _tc_final(ptok, ppe, proj_W, proj_b, ln_scale, ln_bias, proj2_W, proj2_b):
    return pl.pallas_call(
        _tc_final_body,
        grid=(_FIN_GRID,),
        in_specs=[
            pl.BlockSpec((_FIN_ROWS, EMBED), lambda i: (i, 0)),
            pl.BlockSpec((_FIN_ROWS, EMBED), lambda i: (i, 0)),
            pl.BlockSpec((EMBED, OUT), lambda i: (0, 0)),
            pl.BlockSpec((1, OUT), lambda i: (0, 0)),
            pl.BlockSpec((1, OUT), lambda i: (0, 0)),
            pl.BlockSpec((1, OUT), lambda i: (0, 0)),
            pl.BlockSpec((OUT, OUT), lambda i: (0, 0)),
            pl.BlockSpec((1, OUT), lambda i: (0, 0)),
        ],
        out_specs=pl.BlockSpec((_FIN_ROWS, OUT), lambda i: (i, 0)),
        out_shape=jax.ShapeDtypeStruct((B, OUT), jnp.float32),
    )(ptok, ppe, proj_W, proj_b, ln_scale, ln_bias, proj2_W, proj2_b)


def kernel(tokens, token_embed, pos_embed, attn_W, attn_b, proj_W, proj_b,
           ln_scale, ln_bias, proj2_W, proj2_b):
    aw_row = attn_W.reshape(1, EMBED)
    ts = _tc_scores(token_embed, aw_row)
    rawts = _sc_score_gather(ts, tokens.reshape(B * SEQ)).reshape(B, SEQ)
    w, pooled_pe = _tc_mid(rawts, pos_embed, aw_row)
    pooled_tok = _sc_pool(token_embed, tokens, w)
    return _tc_final(pooled_tok, pooled_pe, proj_W, proj_b.reshape(1, OUT),
                     ln_scale.reshape(1, OUT), ln_bias.reshape(1, OUT),
                     proj2_W, proj2_b.reshape(1, OUT))
